# Initial kernel scaffold; baseline (speedup 1.0000x reference)
import jax
import jax.numpy as jnp
from jax.experimental import pallas as pl


def kernel(inputs, clause_weight):
    raise NotImplementedError("write your pallas kernel here")


# speedup vs baseline: 5.5877x; 5.5877x over previous
import functools

import jax
import jax.numpy as jnp
import numpy as np
from jax.experimental import pallas as pl
from jax.experimental.pallas import tpu as pltpu

_L = 8
_BLK = 2048
_SIGNS = np.array([-1.0, 1.0, -1.0, 1.0, -1.0, 1.0, -1.0, 1.0], dtype=np.float32)


def _body(w_ref, x_ref, o_ref):
    lane = jax.lax.broadcasted_iota(jnp.int32, (1, _L), 1)
    signs = jnp.where(lane % 2 == 0, -1.0, 1.0)
    x = x_ref[:, 0:_L]
    cm = x * signs
    m = jnp.max(cm, axis=-1, keepdims=True)
    e = jnp.exp(cm - m)
    sm = e / jnp.sum(e, axis=-1, keepdims=True)
    delta = sm * signs * w_ref[0]
    blk = x_ref.shape[0]
    o_ref[...] = jnp.concatenate(
        [delta, jnp.zeros((blk, o_ref.shape[1] - _L), jnp.float32)], axis=1)


@jax.jit
def kernel(inputs, clause_weight):
    b, p = inputs.shape
    grid = (b // _BLK,)
    out = pl.pallas_call(
        _body,
        grid=grid,
        in_specs=[
            pl.BlockSpec(memory_space=pltpu.SMEM),
            pl.BlockSpec((_BLK, 128), lambda i: (i, 0)),
        ],
        out_specs=pl.BlockSpec((_BLK, p), lambda i: (i, 0)),
        out_shape=jax.ShapeDtypeStruct((b, p), jnp.float32),
    )(clause_weight.reshape(1), inputs)
    return out
